# baseline (speedup 1.0000x reference)
import functools

import jax
import jax.numpy as jnp
from jax import lax
from jax.experimental import pallas as pl
from jax.experimental.pallas import tpu as pltpu
from jax.experimental.pallas import tpu_sc as plsc

N = 10000
NP = 10240
E = 320000
NC = 2
NS = 16
NW = NC * NS
EPW = E // NW
LANES = 16
CHUNKS = EPW // LANES

_SC_MESH = dict(core_axis_name="c", subcore_axis_name="s",
                num_cores=NC, num_subcores=NS)
_SC_PARAMS = pltpu.CompilerParams(needs_layout_passes=False,
                                  use_tc_tiling_on_sc=False)


def _wid():
    return lax.axis_index("s") * NC + lax.axis_index("c")


EWIN = EPW + 240


def _deg_body(ei_hbm, out_hbm, src_out, dst_out, ei_v, srcl_v, dstl_v, acc_v):
    w = _wid()
    a = jnp.minimum(w * EPW // 128 * 128, E - EWIN)
    off = w * EPW - a
    pltpu.sync_copy(ei_hbm.at[:, pl.ds(a, EWIN)], ei_v)
    one = jnp.ones((LANES,), jnp.float32)
    zero = jnp.zeros((LANES,), jnp.float32)

    @plsc.parallel_loop(0, NP // LANES, unroll=8)
    def _(i):
        acc_v[pl.ds(i * LANES, LANES)] = zero

    @plsc.parallel_loop(0, CHUNKS, unroll=8)
    def _(i):
        s = ei_v[0, pl.ds(off + i * LANES, LANES)]
        d = ei_v[1, pl.ds(off + i * LANES, LANES)]
        srcl_v[pl.ds(i * LANES, LANES)] = s
        dstl_v[pl.ds(i * LANES, LANES)] = d
        plsc.addupdate_scatter(acc_v, [d], one)

    pltpu.sync_copy(acc_v, out_hbm.at[pl.ds(w * NP, NP)])
    pltpu.sync_copy(srcl_v, src_out.at[pl.ds(w * NP, EPW)])
    pltpu.sync_copy(dstl_v, dst_out.at[pl.ds(w * NP, EPW)])


_deg_kernel = functools.partial(
    pl.kernel,
    out_type=(
        jax.ShapeDtypeStruct((NW * NP,), jnp.float32),
        jax.ShapeDtypeStruct((NW * NP,), jnp.int32),
        jax.ShapeDtypeStruct((NW * NP,), jnp.int32),
    ),
    mesh=plsc.VectorSubcoreMesh(**_SC_MESH),
    compiler_params=pltpu.CompilerParams(needs_layout_passes=False,
                                         use_tc_tiling_on_sc=True),
    scratch_types=[
        pltpu.VMEM((2, EWIN), jnp.int32),
        pltpu.VMEM((EPW,), jnp.int32),
        pltpu.VMEM((EPW,), jnp.int32),
        pltpu.VMEM((NP,), jnp.float32),
    ],
)(_deg_body)


def _agg_body(k_planes, *refs):
    hwt_hbm, src_hbm, dst_hbm = refs[:3]
    outs = refs[3:3 + k_planes]
    src_v, dst_v = refs[3 + k_planes:5 + k_planes]
    hw_vs = refs[5 + k_planes:5 + 2 * k_planes]
    acc_vs = refs[5 + 2 * k_planes:5 + 3 * k_planes]
    w = _wid()
    pltpu.sync_copy(src_hbm.at[pl.ds(w * NP, EPW)], src_v)
    pltpu.sync_copy(dst_hbm.at[pl.ds(w * NP, EPW)], dst_v)
    for k in range(k_planes):
        pltpu.sync_copy(hwt_hbm.at[pl.ds(k * NP, NP)], hw_vs[k])

    zero = jnp.zeros((LANES,), jnp.float32)

    @plsc.parallel_loop(0, NP // LANES, unroll=8)
    def _(i):
        for k in range(k_planes):
            acc_vs[k][pl.ds(i * LANES, LANES)] = zero

    @plsc.parallel_loop(0, CHUNKS, unroll=8)
    def _(i):
        sl = pl.ds(i * LANES, LANES)
        vs = src_v[sl]
        vd = dst_v[sl]
        for k in range(k_planes):
            g = plsc.load_gather(hw_vs[k], [vs])
            plsc.addupdate_scatter(acc_vs[k], [vd], g)

    for k in range(k_planes):
        pltpu.sync_copy(acc_vs[k], outs[k].at[pl.ds(w * NP, NP)])


def _make_agg_kernel(k_planes):
    return functools.partial(
        pl.kernel,
        out_type=tuple(jax.ShapeDtypeStruct((NW * NP,), jnp.float32)
                       for _ in range(k_planes)),
        mesh=plsc.VectorSubcoreMesh(**_SC_MESH),
        compiler_params=_SC_PARAMS,
        scratch_types=(
            [pltpu.VMEM((EPW,), jnp.int32)] * 2
            + [pltpu.VMEM((NP,), jnp.float32)] * (2 * k_planes)
        ),
    )(functools.partial(_agg_body, k_planes))


_agg4 = _make_agg_kernel(4)
_agg2 = _make_agg_kernel(2)


def _prep_body(dp_ref, x_ref, w1T_ref, dinv_ref, hwt1_ref):
    deg = jnp.ones((NP,), jnp.float32)
    for r in range(NW):
        deg = deg + dp_ref[pl.ds(r * NP, NP)]
    dinv = lax.rsqrt(deg)
    dinv_ref[...] = dinv
    hw1p = lax.dot_general(w1T_ref[...], x_ref[...],
                           (((1,), (1,)), ((), ())),
                           preferred_element_type=jnp.float32)
    zpad = jnp.zeros((NP - N,), jnp.float32)
    for k in range(4):
        row = jnp.concatenate([hw1p[k], zpad])
        hwt1_ref[pl.ds(k * NP, NP)] = dinv * row


def _tc_prep(deg_partials, x, W1T):
    return pl.pallas_call(
        _prep_body,
        out_shape=(
            jax.ShapeDtypeStruct((NP,), jnp.float32),
            jax.ShapeDtypeStruct((4 * NP,), jnp.float32),
        ),
    )(deg_partials, x, W1T)


def _dense_body(k_in, k_out, refs):
    ps = refs[:k_in]
    hwt_ref, dinv_ref, b_ref, wT_ref = refs[k_in:k_in + 4]
    hwtn_ref = refs[k_in + 4]
    dinv = dinv_ref[...]
    hs = []
    for k in range(k_in):
        agg = ps[k][pl.ds(0, NP)]
        for r in range(1, NW):
            agg = agg + ps[k][pl.ds(r * NP, NP)]
        agg = agg + hwt_ref[pl.ds(k * NP, NP)]
        hs.append(jnp.tanh(dinv * agg + b_ref[0, k]))
    for j in range(k_out):
        acc = wT_ref[j, 0] * hs[0]
        for k in range(1, k_in):
            acc = acc + wT_ref[j, k] * hs[k]
        hwtn_ref[pl.ds(j * NP, NP)] = dinv * acc


def _tc_dense(k_in, k_out, partials, hwt, dinv, b, WT):
    smem = pl.BlockSpec(memory_space=pltpu.SMEM)
    body = lambda *refs: _dense_body(k_in, k_out, refs)
    return pl.pallas_call(
        body,
        in_specs=[pl.BlockSpec() for _ in partials]
                 + [pl.BlockSpec(), pl.BlockSpec(), smem, smem],
        out_shape=jax.ShapeDtypeStruct((k_out * NP,), jnp.float32),
    )(*partials, hwt, dinv, b, WT)


def _final_body(refs):
    p0, p1, hwt_ref, dinv_ref, b_ref, wcT_ref, bc_ref, h3_ref, out_ref = refs
    dinv = dinv_ref[...]
    hs = []
    for k in range(2):
        pk = (p0, p1)[k]
        agg = pk[pl.ds(0, NP)]
        for r in range(1, NW):
            agg = agg + pk[pl.ds(r * NP, NP)]
        agg = agg + hwt_ref[pl.ds(k * NP, NP)]
        h = jnp.tanh(dinv * agg + b_ref[0, k])
        hs.append(h)
        h3_ref[k, :] = h[:N]
    for j in range(10):
        out_ref[j, :] = (wcT_ref[j, 0] * hs[0] + wcT_ref[j, 1] * hs[1])[:N] \
            + bc_ref[0, j]


def _tc_final(p3, hwt3, dinv, b3, WcT, bc):
    smem = pl.BlockSpec(memory_space=pltpu.SMEM)
    return pl.pallas_call(
        lambda *refs: _final_body(refs),
        in_specs=[pl.BlockSpec(), pl.BlockSpec(), pl.BlockSpec(),
                  pl.BlockSpec(), smem, smem, smem],
        out_shape=(
            jax.ShapeDtypeStruct((2, N), jnp.float32),
            jax.ShapeDtypeStruct((10, N), jnp.float32),
        ),
    )(*p3, hwt3, dinv, b3, WcT, bc)


def kernel(x, edge_index, W1, b1, W2, b2, W3, b3, Wc, bc):
    deg_partials, src_lin, dst_lin = _deg_kernel(edge_index)
    dinv, hwt1 = _tc_prep(deg_partials, x, W1.T)

    p1 = _agg4(hwt1, src_lin, dst_lin)
    hwt2 = _tc_dense(4, 4, p1, hwt1, dinv, b1.reshape(1, 4), W2.T)

    p2 = _agg4(hwt2, src_lin, dst_lin)
    hwt3 = _tc_dense(4, 2, p2, hwt2, dinv, b2.reshape(1, 4), W3.T)

    p3 = _agg2(hwt3, src_lin, dst_lin)
    h3, outp = _tc_final(p3, hwt3, dinv, b3.reshape(1, 2), Wc.T,
                         bc.reshape(1, 10))
    return (outp.T, h3.T)


# speedup vs baseline: 1.0501x; 1.0501x over previous
import functools

import jax
import jax.numpy as jnp
from jax import lax
from jax.experimental import pallas as pl
from jax.experimental.pallas import tpu as pltpu
from jax.experimental.pallas import tpu_sc as plsc

N = 10000
NP = 10240
E = 320000
NC = 2
NS = 16
NW = NC * NS
EPW = E // NW
LANES = 16
CHUNKS = EPW // LANES

_SC_MESH = dict(core_axis_name="c", subcore_axis_name="s",
                num_cores=NC, num_subcores=NS)
_SC_PARAMS = pltpu.CompilerParams(needs_layout_passes=False,
                                  use_tc_tiling_on_sc=False)


def _wid():
    return lax.axis_index("s") * NC + lax.axis_index("c")


EWIN = EPW + 240


def _edge_window(ei_hbm, ei_v):
    w = _wid()
    a = jnp.minimum(w * EPW // 128 * 128, E - EWIN)
    pltpu.sync_copy(ei_hbm.at[:, pl.ds(a, EWIN)], ei_v)
    return w * EPW - a


def _deg_body(ei_hbm, out_hbm, ei_v, acc_v):
    w = _wid()
    off = _edge_window(ei_hbm, ei_v)
    one = jnp.ones((LANES,), jnp.float32)
    zero = jnp.zeros((LANES,), jnp.float32)

    @plsc.parallel_loop(0, NP // LANES, unroll=8)
    def _(i):
        acc_v[pl.ds(i * LANES, LANES)] = zero

    @plsc.parallel_loop(0, CHUNKS, unroll=8)
    def _(i):
        d = ei_v[1, pl.ds(off + i * LANES, LANES)]
        plsc.addupdate_scatter(acc_v, [d], one)

    pltpu.sync_copy(acc_v, out_hbm.at[pl.ds(w * NP, NP)])


_SC_PARAMS_T = pltpu.CompilerParams(needs_layout_passes=False,
                                    use_tc_tiling_on_sc=True)

_deg_kernel = functools.partial(
    pl.kernel,
    out_type=jax.ShapeDtypeStruct((NW * NP,), jnp.float32),
    mesh=plsc.VectorSubcoreMesh(**_SC_MESH),
    compiler_params=_SC_PARAMS_T,
    scratch_types=[
        pltpu.VMEM((2, EWIN), jnp.int32),
        pltpu.VMEM((NP,), jnp.float32),
    ],
)(_deg_body)


def _agg_body(k_planes, *refs):
    hwt_hbm, ei_hbm = refs[:2]
    outs = refs[2:2 + k_planes]
    ei_v = refs[2 + k_planes]
    hw_vs = refs[3 + k_planes:3 + 2 * k_planes]
    acc_vs = refs[3 + 2 * k_planes:3 + 3 * k_planes]
    w = _wid()
    off = _edge_window(ei_hbm, ei_v)
    for k in range(k_planes):
        pltpu.sync_copy(hwt_hbm.at[pl.ds(k * NP, NP)], hw_vs[k])

    zero = jnp.zeros((LANES,), jnp.float32)

    @plsc.parallel_loop(0, NP // LANES, unroll=8)
    def _(i):
        for k in range(k_planes):
            acc_vs[k][pl.ds(i * LANES, LANES)] = zero

    @plsc.parallel_loop(0, CHUNKS, unroll=8)
    def _(i):
        sl = pl.ds(off + i * LANES, LANES)
        vs = ei_v[0, sl]
        vd = ei_v[1, sl]
        for k in range(k_planes):
            g = plsc.load_gather(hw_vs[k], [vs])
            plsc.addupdate_scatter(acc_vs[k], [vd], g)

    for k in range(k_planes):
        pltpu.sync_copy(acc_vs[k], outs[k].at[pl.ds(w * NP, NP)])


def _make_agg_kernel(k_planes):
    return functools.partial(
        pl.kernel,
        out_type=tuple(jax.ShapeDtypeStruct((NW * NP,), jnp.float32)
                       for _ in range(k_planes)),
        mesh=plsc.VectorSubcoreMesh(**_SC_MESH),
        compiler_params=_SC_PARAMS_T,
        scratch_types=(
            [pltpu.VMEM((2, EWIN), jnp.int32)]
            + [pltpu.VMEM((NP,), jnp.float32)] * (2 * k_planes)
        ),
    )(functools.partial(_agg_body, k_planes))


_agg4 = _make_agg_kernel(4)
_agg2 = _make_agg_kernel(2)


def _prep_body(dp_ref, x_ref, w1T_ref, dinv_ref, hwt1_ref):
    deg = jnp.ones((NP,), jnp.float32)
    for r in range(NW):
        deg = deg + dp_ref[pl.ds(r * NP, NP)]
    dinv = lax.rsqrt(deg)
    dinv_ref[...] = dinv
    hw1p = lax.dot_general(w1T_ref[...], x_ref[...],
                           (((1,), (1,)), ((), ())),
                           preferred_element_type=jnp.float32)
    zpad = jnp.zeros((NP - N,), jnp.float32)
    for k in range(4):
        row = jnp.concatenate([hw1p[k], zpad])
        hwt1_ref[pl.ds(k * NP, NP)] = dinv * row


def _tc_prep(deg_partials, x, W1T):
    return pl.pallas_call(
        _prep_body,
        out_shape=(
            jax.ShapeDtypeStruct((NP,), jnp.float32),
            jax.ShapeDtypeStruct((4 * NP,), jnp.float32),
        ),
    )(deg_partials, x, W1T)


def _dense_body(k_in, k_out, refs):
    ps = refs[:k_in]
    hwt_ref, dinv_ref, b_ref, wT_ref = refs[k_in:k_in + 4]
    hwtn_ref = refs[k_in + 4]
    dinv = dinv_ref[...]
    hs = []
    for k in range(k_in):
        agg = ps[k][pl.ds(0, NP)]
        for r in range(1, NW):
            agg = agg + ps[k][pl.ds(r * NP, NP)]
        agg = agg + hwt_ref[pl.ds(k * NP, NP)]
        hs.append(jnp.tanh(dinv * agg + b_ref[0, k]))
    for j in range(k_out):
        acc = wT_ref[j, 0] * hs[0]
        for k in range(1, k_in):
            acc = acc + wT_ref[j, k] * hs[k]
        hwtn_ref[pl.ds(j * NP, NP)] = dinv * acc


def _tc_dense(k_in, k_out, partials, hwt, dinv, b, WT):
    smem = pl.BlockSpec(memory_space=pltpu.SMEM)
    body = lambda *refs: _dense_body(k_in, k_out, refs)
    return pl.pallas_call(
        body,
        in_specs=[pl.BlockSpec() for _ in partials]
                 + [pl.BlockSpec(), pl.BlockSpec(), smem, smem],
        out_shape=jax.ShapeDtypeStruct((k_out * NP,), jnp.float32),
    )(*partials, hwt, dinv, b, WT)


def _final_body(refs):
    p0, p1, hwt_ref, dinv_ref, b_ref, wcT_ref, bc_ref, h3_ref, out_ref = refs
    dinv = dinv_ref[...]
    hs = []
    for k in range(2):
        pk = (p0, p1)[k]
        agg = pk[pl.ds(0, NP)]
        for r in range(1, NW):
            agg = agg + pk[pl.ds(r * NP, NP)]
        agg = agg + hwt_ref[pl.ds(k * NP, NP)]
        h = jnp.tanh(dinv * agg + b_ref[0, k])
        hs.append(h)
        h3_ref[k, :] = h[:N]
    for j in range(10):
        out_ref[j, :] = (wcT_ref[j, 0] * hs[0] + wcT_ref[j, 1] * hs[1])[:N] \
            + bc_ref[0, j]


def _tc_final(p3, hwt3, dinv, b3, WcT, bc):
    smem = pl.BlockSpec(memory_space=pltpu.SMEM)
    return pl.pallas_call(
        lambda *refs: _final_body(refs),
        in_specs=[pl.BlockSpec(), pl.BlockSpec(), pl.BlockSpec(),
                  pl.BlockSpec(), smem, smem, smem],
        out_shape=(
            jax.ShapeDtypeStruct((2, N), jnp.float32),
            jax.ShapeDtypeStruct((10, N), jnp.float32),
        ),
    )(*p3, hwt3, dinv, b3, WcT, bc)


def kernel(x, edge_index, W1, b1, W2, b2, W3, b3, Wc, bc):
    deg_partials = _deg_kernel(edge_index)
    dinv, hwt1 = _tc_prep(deg_partials, x, W1.T)

    p1 = _agg4(hwt1, edge_index)
    hwt2 = _tc_dense(4, 4, p1, hwt1, dinv, b1.reshape(1, 4), W2.T)

    p2 = _agg4(hwt2, edge_index)
    hwt3 = _tc_dense(4, 2, p2, hwt2, dinv, b2.reshape(1, 4), W3.T)

    p3 = _agg2(hwt3, edge_index)
    h3, outp = _tc_final(p3, hwt3, dinv, b3.reshape(1, 2), Wc.T,
                         bc.reshape(1, 10))
    return (outp.T, h3.T)
